# baseline (speedup 1.0000x reference)
import functools

import jax
import jax.numpy as jnp
from jax import lax
from jax.experimental import pallas as pl
from jax.experimental.pallas import tpu as pltpu
from jax.experimental.pallas import tpu_sc as plsc

NC = 2
NS = 16
LANES = 16
CHUNK = 96


def _deg_hist(row_p, n_pad):
  per_w = row_p.shape[0] // (NC * NS)
  n_chunks = per_w // CHUNK
  slab = n_pad // NS
  assert per_w % CHUNK == 0
  assert n_pad % NS == 0 and slab % LANES == 0

  mesh = plsc.VectorSubcoreMesh(core_axis_name="c", subcore_axis_name="s")

  @functools.partial(
      pl.kernel,
      out_type=jax.ShapeDtypeStruct((NC, n_pad), jnp.float32),
      mesh=mesh,
      scratch_types=[
          pltpu.VMEM((CHUNK,), jnp.int32),
          pltpu.VMEM((CHUNK,), jnp.float32),
          pltpu.VMEM((slab,), jnp.float32),
          pltpu.VMEM_SHARED((n_pad,), jnp.float32),
      ],
  )
  def k(row_hbm, out_hbm, ridx, ones, zbuf, acc):
    c = lax.axis_index("c")
    s = lax.axis_index("s")
    wid = c * NS + s

    @pl.loop(0, slab // LANES)
    def _(i):
      zbuf[pl.ds(i * LANES, LANES)] = jnp.zeros((LANES,), jnp.float32)

    @pl.loop(0, CHUNK // LANES)
    def _(i):
      ones[pl.ds(i * LANES, LANES)] = jnp.ones((LANES,), jnp.float32)

    pltpu.sync_copy(zbuf, acc.at[pl.ds(s * slab, slab)])
    plsc.subcore_barrier()

    @pl.loop(0, n_chunks)
    def _(j):
      pltpu.sync_copy(row_hbm.at[pl.ds(wid * per_w + j * CHUNK, CHUNK)],
                      ridx)
      pltpu.sync_copy(ones, acc.at[ridx], add=True)

    plsc.subcore_barrier()
    pltpu.sync_copy(acc.at[pl.ds(s * slab, slab)],
                    out_hbm.at[c].at[pl.ds(s * slab, slab)])

  return k(row_p)


def _seg_rows(g, row_p, col_p):
  n_pad, d = g.shape
  per_w = row_p.shape[0] // (NC * NS)
  n_chunks = per_w // CHUNK
  assert per_w % CHUNK == 0 and n_chunks % 2 == 0
  slab = n_pad // NS
  zfull, ztail = slab // CHUNK, slab % CHUNK
  assert n_pad % NS == 0 and slab % 8 == 0 and ztail % 8 == 0

  mesh = plsc.VectorSubcoreMesh(core_axis_name="c", subcore_axis_name="s")

  @functools.partial(
      pl.kernel,
      out_type=jax.ShapeDtypeStruct((NC, n_pad, d), jnp.float32),
      mesh=mesh,
      scratch_types=[
          pltpu.VMEM((per_w,), jnp.int32),
          pltpu.VMEM((CHUNK,), jnp.int32),
          pltpu.VMEM((CHUNK,), jnp.int32),
          pltpu.VMEM((CHUNK, d), jnp.float32),
          pltpu.VMEM((CHUNK, d), jnp.float32),
          pltpu.VMEM_SHARED((n_pad, d), jnp.float32),
          pltpu.SemaphoreType.DMA,
          pltpu.SemaphoreType.DMA,
          pltpu.SemaphoreType.DMA,
          pltpu.SemaphoreType.DMA,
      ],
  )
  def k(g_hbm, row_hbm, col_hbm, out_hbm, cidx, rbuf0, rbuf1, buf0, buf1,
        acc, gs0, gs1, rs0, rs1):
    c = lax.axis_index("c")
    s = lax.axis_index("s")
    wid = c * NS + s
    base = wid * per_w

    def load_rows(j, rbuf, sem):
      pltpu.async_copy(row_hbm.at[pl.ds(base + j * CHUNK, CHUNK)], rbuf,
                       sem)

    def wait_rows(j, rbuf, sem):
      pltpu.make_async_copy(row_hbm.at[pl.ds(base + j * CHUNK, CHUNK)],
                            rbuf, sem).wait()

    def gather(j, buf, sem):
      pltpu.async_copy(g_hbm.at[cidx.at[pl.ds(j * CHUNK, CHUNK)]], buf,
                       sem)

    def wait_gather(j, buf, sem):
      pltpu.make_async_copy(g_hbm.at[cidx.at[pl.ds(j * CHUNK, CHUNK)]],
                            buf, sem).wait()

    @pl.loop(0, CHUNK)
    def _(i):
      @pl.loop(0, d // LANES)
      def _(j):
        buf0[i, pl.ds(j * LANES, LANES)] = jnp.zeros((LANES,), jnp.float32)

    pltpu.async_copy(col_hbm.at[pl.ds(base, per_w)], cidx, gs0)

    @pl.loop(0, zfull)
    def _(i):
      pltpu.sync_copy(buf0, acc.at[pl.ds(s * slab + i * CHUNK, CHUNK)])

    if ztail:
      pltpu.sync_copy(buf0.at[pl.ds(0, ztail)],
                      acc.at[pl.ds(s * slab + zfull * CHUNK, ztail)])

    pltpu.make_async_copy(col_hbm.at[pl.ds(base, per_w)], cidx, gs0).wait()
    plsc.subcore_barrier()

    load_rows(0, rbuf0, rs0)
    load_rows(1, rbuf1, rs1)
    gather(0, buf0, gs0)
    gather(1, buf1, gs1)

    @pl.loop(0, n_chunks, step=2)
    def _(j):
      wait_gather(j, buf0, gs0)
      wait_rows(j, rbuf0, rs0)
      pltpu.sync_copy(buf0, acc.at[rbuf0], add=True)

      @pl.when(j + 2 < n_chunks)
      def _():
        load_rows(j + 2, rbuf0, rs0)
        gather(j + 2, buf0, gs0)

      wait_gather(j + 1, buf1, gs1)
      wait_rows(j + 1, rbuf1, rs1)
      pltpu.sync_copy(buf1, acc.at[rbuf1], add=True)

      @pl.when(j + 3 < n_chunks)
      def _():
        load_rows(j + 3, rbuf1, rs1)
        gather(j + 3, buf1, gs1)

    plsc.subcore_barrier()
    pltpu.sync_copy(acc.at[pl.ds(s * slab, slab)],
                    out_hbm.at[c].at[pl.ds(s * slab, slab)])

  return k(g, row_p, col_p)


def _mm_body(x_ref, w_ref, o_ref):
  o_ref[...] = jnp.dot(x_ref[...], w_ref[...],
                       preferred_element_type=jnp.float32)


def _matmul(x, w, blk):
  n, d = x.shape
  return pl.pallas_call(
      _mm_body,
      grid=(n // blk,),
      in_specs=[
          pl.BlockSpec((blk, d), lambda i: (i, 0)),
          pl.BlockSpec((d, d), lambda i: (0, 0)),
      ],
      out_specs=pl.BlockSpec((blk, d), lambda i: (i, 0)),
      out_shape=jax.ShapeDtypeStruct((n, d), jnp.float32),
  )(x, w)


def _scale_body(degp_ref, xw_ref, dinv_ref, g_ref):
  deg = 1.0 + degp_ref[0] + degp_ref[1]
  dinv = lax.rsqrt(deg)
  dinv_ref[...] = dinv
  g_ref[...] = dinv * xw_ref[...]


def _scale(degp, xw, blk):
  n, d = xw.shape
  return pl.pallas_call(
      _scale_body,
      grid=(n // blk,),
      in_specs=[
          pl.BlockSpec((NC, blk, 1), lambda i: (0, i, 0)),
          pl.BlockSpec((blk, d), lambda i: (i, 0)),
      ],
      out_specs=[
          pl.BlockSpec((blk, 1), lambda i: (i, 0)),
          pl.BlockSpec((blk, d), lambda i: (i, 0)),
      ],
      out_shape=[
          jax.ShapeDtypeStruct((n, 1), jnp.float32),
          jax.ShapeDtypeStruct((n, d), jnp.float32),
      ],
  )(degp, xw)


def _mid_body(accp_ref, g_ref, dinv_ref, w_ref, g2_ref):
  dinv = dinv_ref[...]
  h = accp_ref[0] + accp_ref[1] + g_ref[...]
  h = jnp.maximum(dinv * h, 0.0)
  g2_ref[...] = dinv * jnp.dot(h, w_ref[...],
                               preferred_element_type=jnp.float32)


def _mid(accp, g, dinv, w, blk):
  n, d = g.shape
  return pl.pallas_call(
      _mid_body,
      grid=(n // blk,),
      in_specs=[
          pl.BlockSpec((NC, blk, d), lambda i: (0, i, 0)),
          pl.BlockSpec((blk, d), lambda i: (i, 0)),
          pl.BlockSpec((blk, 1), lambda i: (i, 0)),
          pl.BlockSpec((d, d), lambda i: (0, 0)),
      ],
      out_specs=pl.BlockSpec((blk, d), lambda i: (i, 0)),
      out_shape=jax.ShapeDtypeStruct((n, d), jnp.float32),
  )(accp, g, dinv, w)


def _final_body(accp_ref, g_ref, dinv_ref, o_ref):
  dinv = dinv_ref[...]
  h = accp_ref[0] + accp_ref[1] + g_ref[...]
  h = jnp.maximum(dinv * h, 0.0)
  m = jnp.max(h, axis=-1, keepdims=True)
  ex = jnp.exp(h - m)
  o_ref[...] = ex / jnp.sum(ex, axis=-1, keepdims=True)


def _final(accp, g, dinv, blk):
  n, d = g.shape
  return pl.pallas_call(
      _final_body,
      grid=(n // blk,),
      in_specs=[
          pl.BlockSpec((NC, blk, d), lambda i: (0, i, 0)),
          pl.BlockSpec((blk, d), lambda i: (i, 0)),
          pl.BlockSpec((blk, 1), lambda i: (i, 0)),
      ],
      out_specs=pl.BlockSpec((blk, d), lambda i: (i, 0)),
      out_shape=jax.ShapeDtypeStruct((n, d), jnp.float32),
  )(accp, g, dinv)


def kernel(x, edge_index, W0, W1):
  n, d = x.shape
  e = edge_index.shape[1]
  row = edge_index[0]
  col = edge_index[1]

  n_pad = ((n + 1023) // 1024) * 1024
  x_p = jnp.pad(x, ((0, n_pad - n), (0, 0)))

  grp = NC * NS * CHUNK
  n_chunks = ((e + grp - 1) // grp + 1) // 2 * 2
  e_pad = grp * n_chunks
  pad = e_pad - e
  row_p = jnp.concatenate(
      [row, n + (jnp.arange(pad, dtype=jnp.int32) % (n_pad - n))])
  col_p = jnp.concatenate([col, jnp.zeros((pad,), jnp.int32)])

  blk = 1024

  degp = _deg_hist(row_p, n_pad)
  xw0 = _matmul(x_p, W0, blk)
  dinv, g0 = _scale(degp.reshape(NC, n_pad, 1), xw0, blk)
  acc0 = _seg_rows(g0, row_p, col_p)
  g1 = _mid(acc0, g0, dinv, W1, blk)
  acc1 = _seg_rows(g1, row_p, col_p)
  out = _final(acc1, g1, dinv, blk)
  return out[:n]
